# Initial kernel scaffold; baseline (speedup 1.0000x reference)
import jax
import jax.numpy as jnp
from jax.experimental import pallas as pl


def kernel(x, table, W, b):
    raise NotImplementedError("write your pallas kernel here")


# speedup vs baseline: 1.5007x; 1.5007x over previous
import functools

import jax
import jax.numpy as jnp
from jax import lax
from jax.experimental import pallas as pl
from jax.experimental.pallas import tpu as pltpu
from jax.experimental.pallas import tpu_sc as plsc

B = 16384
D = 128

NC = 2
NS = 16
NW = NC * NS

ROWS = 2 * B
ROWS_PER_W = ROWS // NW
CHUNK = 128
N_CHUNKS = ROWS_PER_W // CHUNK


def _sc_gather(table, idx_flat):
    mesh = plsc.VectorSubcoreMesh(core_axis_name="c", subcore_axis_name="s")

    @functools.partial(
        pl.kernel,
        mesh=mesh,
        out_type=jax.ShapeDtypeStruct((ROWS, D), jnp.float32),
        scratch_types=[
            pltpu.VMEM((CHUNK,), jnp.int32),
            pltpu.VMEM((CHUNK, D), jnp.float32),
            pltpu.SemaphoreType.DMA,
        ],
    )
    def gather_kernel(table_hbm, idx_hbm, out_hbm, idx_v, rows_v, sem):
        wid = lax.axis_index("s") * NC + lax.axis_index("c")
        base = wid * ROWS_PER_W
        for c in range(N_CHUNKS):
            off = base + c * CHUNK
            pltpu.sync_copy(idx_hbm.at[pl.ds(off, CHUNK)], idx_v)
            pltpu.async_copy(table_hbm.at[idx_v], rows_v, sem).wait()
            pltpu.sync_copy(rows_v, out_hbm.at[pl.ds(off, CHUNK)])

    return gather_kernel(table, idx_flat)


BLK = 512


def _tc_linear(g, Wt, bias):
    def body(g0_ref, g1_ref, w_ref, b_ref, o_ref):
        acc = jnp.dot(g0_ref[...], w_ref[:D, :],
                      preferred_element_type=jnp.float32)
        acc = acc + jnp.dot(g1_ref[...], w_ref[D:, :],
                            preferred_element_type=jnp.float32)
        o_ref[...] = acc + b_ref[...]

    nblk = B // BLK
    return pl.pallas_call(
        body,
        grid=(nblk,),
        in_specs=[
            pl.BlockSpec((BLK, D), lambda i: (i, 0)),
            pl.BlockSpec((BLK, D), lambda i: (i + nblk, 0)),
            pl.BlockSpec((2 * D, D), lambda i: (0, 0)),
            pl.BlockSpec((1, D), lambda i: (0, 0)),
        ],
        out_specs=pl.BlockSpec((BLK, D), lambda i: (i, 0)),
        out_shape=jax.ShapeDtypeStruct((B, D), jnp.float32),
    )(g, g, Wt, bias)


def kernel(x, table, W, b):
    idx_flat = x.T.reshape(ROWS).astype(jnp.int32)
    g = _sc_gather(table, idx_flat)
    return _tc_linear(g, W.T, b.reshape(1, D))


# speedup vs baseline: 1.7593x; 1.1724x over previous
import functools

import jax
import jax.numpy as jnp
from jax import lax
from jax.experimental import pallas as pl
from jax.experimental.pallas import tpu as pltpu
from jax.experimental.pallas import tpu_sc as plsc

B = 16384
D = 128

NC = 2
NS = 16
NW = NC * NS

ROWS = 2 * B
ROWS_PER_W = ROWS // NW
CHUNK = 128
N_CHUNKS = ROWS_PER_W // CHUNK
NBUF = 7


def _sc_gather(table, idx_grp):
    mesh = plsc.VectorSubcoreMesh(core_axis_name="c", subcore_axis_name="s")

    @functools.partial(
        pl.kernel,
        mesh=mesh,
        out_type=jax.ShapeDtypeStruct((ROWS, D), jnp.float32),
        scratch_types=[
            pltpu.VMEM((N_CHUNKS, CHUNK), jnp.int32),
            pltpu.VMEM((NBUF, CHUNK, D), jnp.float32),
        ]
        + [pltpu.SemaphoreType.DMA] * (2 * NBUF),
    )
    def gather_kernel(table_hbm, idx_hbm, out_hbm, idx_v, rows_v, *sems):
        gsems, wsems = sems[:NBUF], sems[NBUF:]
        wid = lax.axis_index("s") * NC + lax.axis_index("c")
        base = wid * ROWS_PER_W
        pltpu.sync_copy(idx_hbm.at[wid], idx_v)
        gcopy = [None] * N_CHUNKS
        wcopy = [None] * N_CHUNKS
        for c in range(min(NBUF, N_CHUNKS)):
            gcopy[c] = pltpu.async_copy(
                table_hbm.at[idx_v.at[c]], rows_v.at[c], gsems[c])
        for c in range(N_CHUNKS):
            buf = c % NBUF
            gcopy[c].wait()
            wcopy[c] = pltpu.async_copy(
                rows_v.at[buf],
                out_hbm.at[pl.ds(base + c * CHUNK, CHUNK)],
                wsems[buf])
            nxt = c + NBUF
            if nxt < N_CHUNKS:
                wcopy[c].wait()
                gcopy[nxt] = pltpu.async_copy(
                    table_hbm.at[idx_v.at[nxt]], rows_v.at[buf], gsems[buf])
        for c in range(max(0, N_CHUNKS - NBUF), N_CHUNKS):
            wcopy[c].wait()

    return gather_kernel(table, idx_grp)


BLK = 512


def _tc_linear(g, Wt, bias):
    def body(g0_ref, g1_ref, w_ref, b_ref, o_ref):
        acc = jnp.dot(g0_ref[...], w_ref[:D, :],
                      preferred_element_type=jnp.float32)
        acc = acc + jnp.dot(g1_ref[...], w_ref[D:, :],
                            preferred_element_type=jnp.float32)
        o_ref[...] = acc + b_ref[...]

    nblk = B // BLK
    return pl.pallas_call(
        body,
        grid=(nblk,),
        in_specs=[
            pl.BlockSpec((BLK, D), lambda i: (i, 0)),
            pl.BlockSpec((BLK, D), lambda i: (i + nblk, 0)),
            pl.BlockSpec((2 * D, D), lambda i: (0, 0)),
            pl.BlockSpec((1, D), lambda i: (0, 0)),
        ],
        out_specs=pl.BlockSpec((BLK, D), lambda i: (i, 0)),
        out_shape=jax.ShapeDtypeStruct((B, D), jnp.float32),
    )(g, g, Wt, bias)


def kernel(x, table, W, b):
    idx_grp = x.T.reshape(NW, N_CHUNKS, CHUNK).astype(jnp.int32)
    g = _sc_gather(table, idx_grp)
    return _tc_linear(g, W.T, b.reshape(1, D))


# speedup vs baseline: 2.2620x; 1.2857x over previous
import functools

import jax
import jax.numpy as jnp
from jax import lax
from jax.experimental import pallas as pl
from jax.experimental.pallas import tpu as pltpu
from jax.experimental.pallas import tpu_sc as plsc

B = 16384
D = 128

NC = 2
NS = 16
NW = NC * NS

ROWS = 2 * B
ROWS_PER_W = ROWS // NW
CHUNK = 128
N_CHUNKS = ROWS_PER_W // CHUNK
NBUF = 7


def _sc_gather(table, idx_grp):
    mesh = plsc.VectorSubcoreMesh(core_axis_name="c", subcore_axis_name="s")

    @functools.partial(
        pl.kernel,
        mesh=mesh,
        out_type=jax.ShapeDtypeStruct((ROWS, D), jnp.float32),
        scratch_types=[
            pltpu.VMEM((N_CHUNKS, CHUNK), jnp.int32),
            pltpu.VMEM((NBUF, CHUNK, D), jnp.float32),
        ]
        + [pltpu.SemaphoreType.DMA] * (2 * NBUF),
    )
    def gather_kernel(table_hbm, idx_hbm, out_hbm, idx_v, rows_v, *sems):
        gsems, wsems = sems[:NBUF], sems[NBUF:]
        wid = lax.axis_index("s") * NC + lax.axis_index("c")
        base = wid * ROWS_PER_W
        pltpu.sync_copy(idx_hbm.at[wid], idx_v)
        gcopy = [None] * N_CHUNKS
        wcopy = [None] * N_CHUNKS
        for c in range(min(NBUF, N_CHUNKS)):
            gcopy[c] = pltpu.async_copy(
                table_hbm.at[idx_v.at[c]], rows_v.at[c], gsems[c])
        for c in range(N_CHUNKS):
            buf = c % NBUF
            gcopy[c].wait()
            wcopy[c] = pltpu.async_copy(
                rows_v.at[buf],
                out_hbm.at[pl.ds(base + c * CHUNK, CHUNK)],
                wsems[buf])
            nxt = c + NBUF
            if nxt < N_CHUNKS:
                wcopy[c].wait()
                gcopy[nxt] = pltpu.async_copy(
                    table_hbm.at[idx_v.at[nxt]], rows_v.at[buf], gsems[buf])
        for c in range(max(0, N_CHUNKS - NBUF), N_CHUNKS):
            wcopy[c].wait()

    return gather_kernel(table, idx_grp)


BLK = 2048


def _tc_linear(g, W, bias):
    def body(g0_ref, g1_ref, w_ref, b_ref, o_ref):
        dn = (((1,), (1,)), ((), ()))
        acc = lax.dot_general(g0_ref[...], w_ref[:, :D], dn,
                              preferred_element_type=jnp.float32)
        acc = acc + lax.dot_general(g1_ref[...], w_ref[:, D:], dn,
                                    preferred_element_type=jnp.float32)
        o_ref[...] = acc + b_ref[...]

    nblk = B // BLK
    return pl.pallas_call(
        body,
        grid=(nblk,),
        in_specs=[
            pl.BlockSpec((BLK, D), lambda i: (i, 0)),
            pl.BlockSpec((BLK, D), lambda i: (i + nblk, 0)),
            pl.BlockSpec((D, 2 * D), lambda i: (0, 0)),
            pl.BlockSpec((1, D), lambda i: (0, 0)),
        ],
        out_specs=pl.BlockSpec((BLK, D), lambda i: (i, 0)),
        out_shape=jax.ShapeDtypeStruct((B, D), jnp.float32),
    )(g, g, W, bias)


def kernel(x, table, W, b):
    idx_grp = x.T.reshape(NW, N_CHUNKS, CHUNK).astype(jnp.int32)
    g = _sc_gather(table, idx_grp)
    return _tc_linear(g, W, b.reshape(1, D))
